# baseline (speedup 1.0000x reference)
import jax
import jax.numpy as jnp
from jax import lax
from jax.experimental import pallas as pl
from jax.experimental.pallas import tpu as pltpu
from jax.experimental.pallas import tpu_sc as plsc

D = 64
LANES = 128

_info = plsc.get_sparse_core_info()
NC, NS = _info.num_cores, _info.num_subcores
NW = NC * NS

B = 16384 * 50
ROWS = B // LANES
ROWS_PER_W = ROWS // NW
KI = 5
N_OUTER = ROWS_PER_W // KI
CHUNK = KI * LANES


def _gather_body(table_hbm, ids_hbm, out_hbm, idx_v, rows_v, gsem, osem):
    wid = lax.axis_index("s") * NC + lax.axis_index("c")
    row_base = wid * ROWS_PER_W

    def load_idx(c, b):
        pltpu.sync_copy(ids_hbm.at[pl.ds(row_base + c * KI, KI)], idx_v.at[b])

    def fire(c, b):
        for j in range(KI):
            pltpu.async_copy(table_hbm.at[idx_v.at[b, j]],
                             rows_v.at[b, pl.ds(j * LANES, LANES)], gsem.at[b])

    def drain_gather(b):
        for j in range(KI):
            pltpu.make_async_copy(table_hbm.at[idx_v.at[b, j]],
                                  rows_v.at[b, pl.ds(j * LANES, LANES)],
                                  gsem.at[b]).wait()

    def out_copy(c, b):
        return pltpu.make_async_copy(
            rows_v.at[b],
            out_hbm.at[pl.ds((row_base + c * KI) * LANES, CHUNK)],
            osem.at[b])

    load_idx(0, 0)
    fire(0, 0)
    load_idx(1, 1)
    fire(1, 1)

    def step(c, carry):
        b = c % 2
        drain_gather(b)
        out_copy(c, b).start()
        out_copy(c, b).wait()

        @pl.when(c + 2 < N_OUTER)
        def _():
            load_idx(c + 2, b)
            fire(c + 2, b)

        return carry

    lax.fori_loop(0, N_OUTER, step, 0)


@jax.jit
def _embed_lookup(table, ids2d):
    mesh = plsc.VectorSubcoreMesh(core_axis_name="c", subcore_axis_name="s")
    k = pl.kernel(
        _gather_body,
        mesh=mesh,
        out_type=jax.ShapeDtypeStruct((B, D), jnp.float32),
        scratch_types=[
            pltpu.VMEM((2, KI, LANES), jnp.int32),
            pltpu.VMEM((2, CHUNK, D), jnp.float32),
            pltpu.SemaphoreType.DMA((2,)),
            pltpu.SemaphoreType.DMA((2,)),
        ],
        compiler_params=pltpu.CompilerParams(use_tc_tiling_on_sc=False),
    )
    return k(table, ids2d)


def kernel(ids, table):
    ids2d = ids.reshape(ROWS, LANES)
    out = _embed_lookup(table, ids2d)
    return out.reshape(ids.shape[0], ids.shape[1], D)


# speedup vs baseline: 1.4388x; 1.4388x over previous
import jax
import jax.numpy as jnp
from jax import lax
from jax.experimental import pallas as pl
from jax.experimental.pallas import tpu as pltpu
from jax.experimental.pallas import tpu_sc as plsc

D = 64
LANES = 128

_info = plsc.get_sparse_core_info()
NC, NS = _info.num_cores, _info.num_subcores
NW = NC * NS

BATCH = 16384
HIST = 50
NTB = BATCH // LANES
TB_PER_W = NTB // NW
N_BLOCKS = HIST * TB_PER_W


def _gather_body(table_hbm, idsT_hbm, out_hbm,
                 idx_v, rows_v0, rows_v1, trans_v0, trans_v1, gsem, osem):
    wid = lax.axis_index("s") * NC + lax.axis_index("c")
    tb0 = wid * TB_PER_W
    rows_v = (rows_v0, rows_v1)
    trans_v = (trans_v0, trans_v1)

    def coords(g):
        return g // TB_PER_W, tb0 + g % TB_PER_W

    def load_idx(g, b):
        h, tb = coords(g)
        pltpu.sync_copy(idsT_hbm.at[h, pl.ds(tb * LANES, LANES)],
                        idx_v.at[b])

    def gather(g, b):
        return pltpu.make_async_copy(table_hbm.at[idx_v.at[b]],
                                     rows_v[b], gsem.at[b])

    def out_copies(g, b):
        h, tb = coords(g)
        return [
            pltpu.make_async_copy(trans_v[b].at[pl.ds(8 * k, 8)],
                                  out_hbm.at[h, k, tb], osem.at[b])
            for k in range(8)
        ]

    rowvs = [jnp.arange(16, dtype=jnp.int32) + gg * 16 for gg in range(8)]

    def transpose(b):
        @plsc.parallel_loop(0, D, 1, unroll=8)
        def _tbody(o):
            for gg in range(8):
                colv = (rowvs[gg] + o) & 63
                vals = plsc.load_gather(rows_v[b], [rowvs[gg], colv])
                plsc.store_scatter(trans_v[b], [colv, rowvs[gg]], vals)

    load_idx(0, 0)
    gather(0, 0).start()
    load_idx(1, 1)
    gather(1, 1).start()

    def pair(i, carry):
        for b in (0, 1):
            g = 2 * i + b
            gather(g, b).wait()

            @pl.when(g >= 2)
            def _():
                for c in out_copies(g - 2, b):
                    c.wait()

            transpose(b)
            for c in out_copies(g, b):
                c.start()

            @pl.when(g + 2 < N_BLOCKS)
            def _():
                load_idx(g + 2, b)
                gather(g + 2, b).start()

        return carry

    lax.fori_loop(0, N_BLOCKS // 2, pair, 0)

    for c in out_copies(N_BLOCKS - 2, 0):
        c.wait()
    for c in out_copies(N_BLOCKS - 1, 1):
        c.wait()


@jax.jit
def _embed_lookup(table, idsT):
    mesh = plsc.VectorSubcoreMesh(core_axis_name="c", subcore_axis_name="s")
    k = pl.kernel(
        _gather_body,
        mesh=mesh,
        out_type=jax.ShapeDtypeStruct((HIST, 8, NTB, 8, LANES), jnp.float32),
        scratch_types=[
            pltpu.VMEM((2, LANES), jnp.int32),
            pltpu.VMEM((LANES, D), jnp.float32),
            pltpu.VMEM((LANES, D), jnp.float32),
            pltpu.VMEM((D, LANES), jnp.float32),
            pltpu.VMEM((D, LANES), jnp.float32),
            pltpu.SemaphoreType.DMA((2,)),
            pltpu.SemaphoreType.DMA((2,)),
        ],
        compiler_params=pltpu.CompilerParams(use_tc_tiling_on_sc=False,
                                             needs_layout_passes=False),
    )
    return k(table, idsT)


def kernel(ids, table):
    out5 = _embed_lookup(table, ids.T)
    return out5.transpose(2, 4, 0, 1, 3).reshape(BATCH, HIST, D)


# speedup vs baseline: 2.4389x; 1.6951x over previous
import jax
import jax.numpy as jnp
from jax import lax
from jax.experimental import pallas as pl
from jax.experimental.pallas import tpu as pltpu
from jax.experimental.pallas import tpu_sc as plsc

D = 64
LANES = 128

_info = plsc.get_sparse_core_info()
NC, NS = _info.num_cores, _info.num_subcores
NW = NC * NS

BATCH = 16384
HIST = 50
NTB = BATCH // LANES
TB_PER_W = NTB // NW
N_BLOCKS = HIST * TB_PER_W

VOCAB = 1000001
NVT = (VOCAB // LANES)
VPAD = NVT * LANES + 80
VROWS = (NVT + 1) * LANES
A_STEPS = NVT // NW + 1

def _rowv():
    return [jnp.arange(16, dtype=jnp.int32) + gg * 16 for gg in range(8)]


def _fmt_body(tableT_hbm, tail_hbm, out_hbm,
              ctile0, ctile1, trans0, trans1, rsem, wsem):
    wid = lax.axis_index("s") * NC + lax.axis_index("c")
    ctile = (ctile0, ctile1)
    trans = (trans0, trans1)

    def vt_of(g):
        return g * NW + wid

    def reads(g, b):
        vt = vt_of(g)
        return [
            pltpu.make_async_copy(
                tableT_hbm.at[pl.ds(8 * ct, 8), pl.ds(LANES * vt, LANES)],
                ctile[b].at[pl.ds(8 * ct, 8)], rsem.at[b])
            for ct in range(8)
        ]

    def write(g, b):
        return pltpu.make_async_copy(
            trans[b], out_hbm.at[pl.ds(vt_of(g) * 8192, 8192)], wsem.at[b])

    rowvs = _rowv()
    rowx64 = [rowvs[gg] * 64 for gg in range(8)]

    def transpose(b):
        @plsc.parallel_loop(0, D, 1, unroll=8)
        def _tbody(o):
            for gg in range(8):
                colv = (rowvs[gg] + o) & 63
                vals = plsc.load_gather(ctile[b], [colv, rowvs[gg]])
                plsc.store_scatter(trans[b], [rowx64[gg] + colv], vals)

    @pl.when(wid == NW - 1)
    def _():
        pltpu.sync_copy(tail_hbm, trans0.at[pl.ds(0, 5120)])
        pltpu.sync_copy(trans0.at[pl.ds(0, 5120)],
                        out_hbm.at[pl.ds(NVT * 8192, 5120)])

    for c in reads(0, 0):
        c.start()

    @pl.when(vt_of(1) < NVT)
    def _():
        for c in reads(1, 1):
            c.start()

    def pair(i, carry):
        for b in (0, 1):
            g = 2 * i + b

            @pl.when((g >= 2) & (vt_of(g - 2) < NVT))
            def _():
                write(g - 2, b).wait()

            @pl.when(vt_of(g) < NVT)
            def _():
                for c in reads(g, b):
                    c.wait()
                transpose(b)
                write(g, b).start()

            @pl.when(vt_of(g + 2) < NVT)
            def _():
                for c in reads(g + 2, b):
                    c.start()

        return carry

    lax.fori_loop(0, (A_STEPS + 1) // 2, pair, 0)

    for g in (2 * ((A_STEPS + 1) // 2) - 2, 2 * ((A_STEPS + 1) // 2) - 1):
        b = g % 2

        @pl.when(vt_of(g) < NVT)
        def _():
            write(g, b).wait()


def _gather_body(table_hbm, idsT_hbm, out_hbm,
                 idx_v, rows_v0, rows_v1, trans_v0, trans_v1, gsem, osem):
    wid = lax.axis_index("s") * NC + lax.axis_index("c")
    tb0 = wid * TB_PER_W
    rows_v = (rows_v0, rows_v1)
    trans_v = (trans_v0, trans_v1)

    def coords(g):
        return g // TB_PER_W, tb0 + g % TB_PER_W

    def load_idx(g, b):
        h, tb = coords(g)
        pltpu.sync_copy(idsT_hbm.at[h, pl.ds(tb * LANES, LANES)],
                        idx_v.at[b])

    def gather(g, b):
        return pltpu.make_async_copy(table_hbm.at[idx_v.at[b]],
                                     rows_v[b], gsem.at[b])

    def out_copies(g, b):
        h, tb = coords(g)
        return [
            pltpu.make_async_copy(trans_v[b].at[pl.ds(8 * k, 8)],
                                  out_hbm.at[h, k, tb], osem.at[b])
            for k in range(8)
        ]

    rowvs = _rowv()

    def transpose(b):
        @plsc.parallel_loop(0, D, 1, unroll=8)
        def _tbody(o):
            for gg in range(8):
                colv = (rowvs[gg] + o) & 63
                vals = plsc.load_gather(rows_v[b], [rowvs[gg], colv])
                plsc.store_scatter(trans_v[b], [colv, rowvs[gg]], vals)

    load_idx(0, 0)
    gather(0, 0).start()
    load_idx(1, 1)
    gather(1, 1).start()

    def pair(i, carry):
        for b in (0, 1):
            g = 2 * i + b
            gather(g, b).wait()

            @pl.when(g >= 2)
            def _():
                for c in out_copies(g - 2, b):
                    c.wait()

            transpose(b)
            for c in out_copies(g, b):
                c.start()

            @pl.when(g + 2 < N_BLOCKS)
            def _():
                load_idx(g + 2, b)
                gather(g + 2, b).start()

        return carry

    lax.fori_loop(0, N_BLOCKS // 2, pair, 0)

    for c in out_copies(N_BLOCKS - 2, 0):
        c.wait()
    for c in out_copies(N_BLOCKS - 1, 1):
        c.wait()


_MESH = dict(core_axis_name="c", subcore_axis_name="s")


@jax.jit
def _embed_lookup(table, ids):
    fmt = pl.kernel(
        _fmt_body,
        mesh=plsc.VectorSubcoreMesh(**_MESH),
        out_type=jax.ShapeDtypeStruct((VROWS * D,), jnp.float32),
        scratch_types=[
            pltpu.VMEM((D, LANES), jnp.float32),
            pltpu.VMEM((D, LANES), jnp.float32),
            pltpu.VMEM((8192,), jnp.float32),
            pltpu.VMEM((8192,), jnp.float32),
            pltpu.SemaphoreType.DMA((2,)),
            pltpu.SemaphoreType.DMA((2,)),
        ],
        compiler_params=pltpu.CompilerParams(use_tc_tiling_on_sc=True,
                                             needs_layout_passes=False),
    )
    tail = jnp.pad(table[NVT * LANES:], ((0, VPAD - VOCAB), (0, 0)))
    table_rm = fmt(table.T, tail.reshape(-1)).reshape(VROWS, D)

    gat = pl.kernel(
        _gather_body,
        mesh=plsc.VectorSubcoreMesh(**_MESH),
        out_type=jax.ShapeDtypeStruct((HIST, 8, NTB, 8, LANES), jnp.float32),
        scratch_types=[
            pltpu.VMEM((2, LANES), jnp.int32),
            pltpu.VMEM((LANES, D), jnp.float32),
            pltpu.VMEM((LANES, D), jnp.float32),
            pltpu.VMEM((D, LANES), jnp.float32),
            pltpu.VMEM((D, LANES), jnp.float32),
            pltpu.SemaphoreType.DMA((2,)),
            pltpu.SemaphoreType.DMA((2,)),
        ],
        compiler_params=pltpu.CompilerParams(use_tc_tiling_on_sc=False,
                                             needs_layout_passes=False),
    )
    return gat(table_rm, ids.T)


def kernel(ids, table):
    out5 = _embed_lookup(table, ids)
    return out5.transpose(2, 4, 0, 1, 3).reshape(BATCH, HIST, D)
